# baseline (speedup 1.0000x reference)
import functools

import jax
import jax.numpy as jnp
from jax import lax
from jax.experimental import pallas as pl
from jax.experimental.pallas import tpu as pltpu
from jax.experimental.pallas import tpu_sc as plsc

_NC = 2
_NS = 16
_NW = _NC * _NS
_K = 2
_NBUF = 4
_TB = 32


@functools.cache
def _build_gather(batch: int, hist: int, n_emb: int, feat: int):
    rpw = batch // _NW
    nblk = rpw // _K
    assert batch == _NW * rpw and rpw % _K == 0 and n_emb % 2 == 0
    assert nblk % _NBUF == 0 and nblk // _NBUF >= 2 and 16 <= hist <= 128
    mesh = plsc.VectorSubcoreMesh(core_axis_name="c", subcore_axis_name="s")

    @functools.partial(
        pl.kernel,
        out_type=jax.ShapeDtypeStruct((batch, hist, feat), jnp.int32),
        mesh=mesh,
        scratch_types=[
            pltpu.VMEM((rpw, hist), jnp.int32),
            pltpu.VMEM((rpw, hist), jnp.int32),
        ]
        + [pltpu.VMEM((_K, hist, feat), jnp.int32) for _ in range(_NBUF)]
        + [pltpu.SemaphoreType.DMA] * (2 * _NBUF),
        compiler_params=pltpu.CompilerParams(use_tc_tiling_on_sc=True),
    )
    def gather_kernel(table_hbm, idx_hbm, g_hbm, idx_v, rind_v, *rest):
        tab32 = table_hbm.bitcast(jnp.int32)
        gbufs = rest[:_NBUF]
        gsems = rest[_NBUF:2 * _NBUF]
        ssems = rest[2 * _NBUF:]
        wid = lax.axis_index("s") * _NC + lax.axis_index("c")
        row0 = wid * rpw
        pltpu.sync_copy(idx_hbm.at[pl.ds(row0, rpw)], idx_v)

        offs = list(range(0, hist - 15, 16))
        if offs[-1] != hist - 16:
            offs.append(hist - 16)

        def prep_body(b, carry):
            for off in offs:
                rind_v[b, pl.ds(off, 16)] = idx_v[b, pl.ds(off, 16)] >> 1
            return carry

        lax.fori_loop(0, rpw, prep_body, 0)

        def fire(t, b):
            for k in range(_K):
                pltpu.async_copy(
                    tab32.at[rind_v.at[t * _K + k]], gbufs[b].at[k], gsems[b]
                )

        def drain_g(b):
            pltpu.make_async_copy(
                g_hbm.at[pl.ds(0, _K)], gbufs[b], gsems[b]
            ).wait()

        def store_o(t, b):
            pltpu.async_copy(
                gbufs[b], g_hbm.at[pl.ds(row0 + t * _K, _K)], ssems[b]
            )

        def wait_s(b):
            pltpu.make_async_copy(
                gbufs[b], g_hbm.at[pl.ds(0, _K)], ssems[b]
            ).wait()

        for b in range(_NBUF):
            fire(b, b)

        def body(i, carry):
            for b in range(_NBUF):
                drain_g(b)
                store_o(i * _NBUF + b, b)
            for b in range(_NBUF):
                wait_s(b)
                fire(i * _NBUF + b + _NBUF, b)
            return carry

        lax.fori_loop(0, nblk // _NBUF - 1, body, 0)

        t0 = nblk - _NBUF
        for b in range(_NBUF):
            drain_g(b)
            store_o(t0 + b, b)
        for b in range(_NBUF):
            wait_s(b)

    return gather_kernel


def _extract_body(idx_ref, g_ref, o_ref):
    sh = (idx_ref[...] & 1) << 4
    v = (g_ref[...] >> sh[:, :, None]) & 0xFFFF
    o_ref[...] = lax.bitcast_convert_type(
        v.astype(jnp.uint16), jnp.bfloat16
    )


@functools.cache
def _build_extract(batch: int, hist: int, feat: int):
    assert batch % _TB == 0
    return pl.pallas_call(
        _extract_body,
        grid=(batch // _TB,),
        in_specs=[
            pl.BlockSpec((_TB, hist), lambda i: (i, 0)),
            pl.BlockSpec((_TB, hist, feat), lambda i: (i, 0, 0)),
        ],
        out_specs=pl.BlockSpec((_TB, hist, feat), lambda i: (i, 0, 0)),
        out_shape=jax.ShapeDtypeStruct((batch, hist, feat), jnp.bfloat16),
    )


_NCHUNK = 2


def kernel(inputs, embedding):
    batch, hist = inputs.shape
    n_emb, feat = embedding.shape
    idx = inputs.astype(jnp.int32)
    cb = batch // _NCHUNK
    gather = _build_gather(cb, hist, n_emb, feat)
    extract = _build_extract(cb, hist, feat)
    outs = []
    for c in range(_NCHUNK):
        idx_c = lax.slice_in_dim(idx, c * cb, (c + 1) * cb)
        outs.append(extract(idx_c, gather(embedding, idx_c)))
    return jnp.concatenate(outs, axis=0)


# speedup vs baseline: 1.3245x; 1.3245x over previous
import functools

import jax
import jax.numpy as jnp
from jax import lax
from jax.experimental import pallas as pl
from jax.experimental.pallas import tpu as pltpu
from jax.experimental.pallas import tpu_sc as plsc

_NC = 2
_NS = 16
_NW = _NC * _NS
_K = 2
_NBUF = 4
_TB = 64


@functools.cache
def _build_gather(batch: int, hist: int, n_emb: int, feat: int):
    rpw = batch // _NW
    nblk = rpw // _K
    assert batch == _NW * rpw and rpw % _K == 0 and n_emb % 2 == 0
    assert nblk % _NBUF == 0 and nblk // _NBUF >= 2 and 16 <= hist <= 128
    mesh = plsc.VectorSubcoreMesh(core_axis_name="c", subcore_axis_name="s")

    @functools.partial(
        pl.kernel,
        out_type=jax.ShapeDtypeStruct((batch, hist, feat), jnp.int32),
        mesh=mesh,
        scratch_types=[
            pltpu.VMEM((rpw, hist), jnp.int32),
            pltpu.VMEM((rpw, hist), jnp.int32),
        ]
        + [pltpu.VMEM((_K, hist, feat), jnp.int32) for _ in range(_NBUF)]
        + [pltpu.SemaphoreType.DMA] * (2 * _NBUF),
        compiler_params=pltpu.CompilerParams(use_tc_tiling_on_sc=True),
    )
    def gather_kernel(table_hbm, idx_hbm, g_hbm, idx_v, rind_v, *rest):
        tab32 = table_hbm.bitcast(jnp.int32)
        gbufs = rest[:_NBUF]
        gsems = rest[_NBUF:2 * _NBUF]
        ssems = rest[2 * _NBUF:]
        wid = lax.axis_index("s") * _NC + lax.axis_index("c")
        row0 = wid * rpw
        pltpu.sync_copy(idx_hbm.at[pl.ds(row0, rpw)], idx_v)

        offs = list(range(0, hist - 15, 16))
        if offs[-1] != hist - 16:
            offs.append(hist - 16)

        def prep_body(b, carry):
            for off in offs:
                rind_v[b, pl.ds(off, 16)] = idx_v[b, pl.ds(off, 16)] >> 1
            return carry

        lax.fori_loop(0, rpw, prep_body, 0)

        def fire(t, b):
            for k in range(_K):
                pltpu.async_copy(
                    tab32.at[rind_v.at[t * _K + k]], gbufs[b].at[k], gsems[b]
                )

        def drain_g(b):
            pltpu.make_async_copy(
                g_hbm.at[pl.ds(0, _K)], gbufs[b], gsems[b]
            ).wait()

        def store_o(t, b):
            pltpu.async_copy(
                gbufs[b], g_hbm.at[pl.ds(row0 + t * _K, _K)], ssems[b]
            )

        def wait_s(b):
            pltpu.make_async_copy(
                gbufs[b], g_hbm.at[pl.ds(0, _K)], ssems[b]
            ).wait()

        for b in range(_NBUF):
            fire(b, b)

        def body(i, carry):
            for b in range(_NBUF):
                drain_g(b)
                store_o(i * _NBUF + b, b)
            for b in range(_NBUF):
                wait_s(b)
                fire(i * _NBUF + b + _NBUF, b)
            return carry

        lax.fori_loop(0, nblk // _NBUF - 1, body, 0)

        t0 = nblk - _NBUF
        for b in range(_NBUF):
            drain_g(b)
            store_o(t0 + b, b)
        for b in range(_NBUF):
            wait_s(b)

    return gather_kernel


def _extract_body(idx_ref, g_ref, o_ref):
    sh = (idx_ref[...] & 1) << 4
    v = (g_ref[...] >> sh[:, :, None]) & 0xFFFF
    o_ref[...] = lax.bitcast_convert_type(
        v.astype(jnp.uint16), jnp.bfloat16
    )


@functools.cache
def _build_extract(batch: int, hist: int, feat: int):
    assert batch % _TB == 0
    return pl.pallas_call(
        _extract_body,
        grid=(batch // _TB,),
        in_specs=[
            pl.BlockSpec((_TB, hist), lambda i: (i, 0)),
            pl.BlockSpec((_TB, hist, feat), lambda i: (i, 0, 0)),
        ],
        out_specs=pl.BlockSpec((_TB, hist, feat), lambda i: (i, 0, 0)),
        out_shape=jax.ShapeDtypeStruct((batch, hist, feat), jnp.bfloat16),
    )


def kernel(inputs, embedding):
    batch, hist = inputs.shape
    n_emb, feat = embedding.shape
    idx = inputs.astype(jnp.int32)
    g = _build_gather(batch, hist, n_emb, feat)(embedding, idx)
    return _build_extract(batch, hist, feat)(idx, g)


# speedup vs baseline: 1.4521x; 1.0964x over previous
import functools

import jax
import jax.numpy as jnp
from jax import lax
from jax.experimental import pallas as pl
from jax.experimental.pallas import tpu as pltpu
from jax.experimental.pallas import tpu_sc as plsc

_NC = 2
_NS = 16
_NW = _NC * _NS
_K = 2
_NBUF = 4
_TB = 128


@functools.cache
def _build_gather(batch: int, hist: int, n_emb: int, feat: int):
    rpw = batch // _NW
    nblk = rpw // _K
    assert batch == _NW * rpw and rpw % _K == 0 and n_emb % 2 == 0
    assert nblk % _NBUF == 0 and nblk // _NBUF >= 2 and 16 <= hist <= 128
    mesh = plsc.VectorSubcoreMesh(core_axis_name="c", subcore_axis_name="s")

    @functools.partial(
        pl.kernel,
        out_type=jax.ShapeDtypeStruct((batch, hist, feat), jnp.int32),
        mesh=mesh,
        scratch_types=[
            pltpu.VMEM((rpw, hist), jnp.int32),
            pltpu.VMEM((rpw, hist), jnp.int32),
        ]
        + [pltpu.VMEM((_K, hist, feat), jnp.int32) for _ in range(_NBUF)]
        + [pltpu.SemaphoreType.DMA] * (2 * _NBUF),
        compiler_params=pltpu.CompilerParams(use_tc_tiling_on_sc=True),
    )
    def gather_kernel(table_hbm, idx_hbm, g_hbm, idx_v, rind_v, *rest):
        tab32 = table_hbm.bitcast(jnp.int32)
        gbufs = rest[:_NBUF]
        gsems = rest[_NBUF:2 * _NBUF]
        ssems = rest[2 * _NBUF:]
        wid = lax.axis_index("s") * _NC + lax.axis_index("c")
        row0 = wid * rpw
        pltpu.sync_copy(idx_hbm.at[pl.ds(row0, rpw)], idx_v)

        offs = list(range(0, hist - 15, 16))
        if offs[-1] != hist - 16:
            offs.append(hist - 16)

        def prep_body(b, carry):
            for off in offs:
                rind_v[b, pl.ds(off, 16)] = idx_v[b, pl.ds(off, 16)] >> 1
            return carry

        lax.fori_loop(0, rpw, prep_body, 0)

        def fire(t, b):
            for k in range(_K):
                pltpu.async_copy(
                    tab32.at[rind_v.at[t * _K + k]], gbufs[b].at[k], gsems[b]
                )

        def drain_g(b):
            pltpu.make_async_copy(
                g_hbm.at[pl.ds(0, _K)], gbufs[b], gsems[b]
            ).wait()

        def store_o(t, b):
            pltpu.async_copy(
                gbufs[b], g_hbm.at[pl.ds(row0 + t * _K, _K)], ssems[b]
            )

        def wait_s(b):
            pltpu.make_async_copy(
                gbufs[b], g_hbm.at[pl.ds(0, _K)], ssems[b]
            ).wait()

        for b in range(_NBUF):
            fire(b, b)

        def body(i, carry):
            for b in range(_NBUF):
                drain_g(b)
                store_o(i * _NBUF + b, b)
            for b in range(_NBUF):
                wait_s(b)
                fire(i * _NBUF + b + _NBUF, b)
            return carry

        lax.fori_loop(0, nblk // _NBUF - 1, body, 0)

        t0 = nblk - _NBUF
        for b in range(_NBUF):
            drain_g(b)
            store_o(t0 + b, b)
        for b in range(_NBUF):
            wait_s(b)

    return gather_kernel


def _extract_body(idx_ref, g_ref, o_ref):
    sh = (idx_ref[...] & 1) << 4
    v = (g_ref[...] >> sh[:, :, None]) & 0xFFFF
    o_ref[...] = lax.bitcast_convert_type(
        v.astype(jnp.uint16), jnp.bfloat16
    )


@functools.cache
def _build_extract(batch: int, hist: int, feat: int):
    assert batch % _TB == 0
    return pl.pallas_call(
        _extract_body,
        grid=(batch // _TB,),
        in_specs=[
            pl.BlockSpec((_TB, hist), lambda i: (i, 0)),
            pl.BlockSpec((_TB, hist, feat), lambda i: (i, 0, 0)),
        ],
        out_specs=pl.BlockSpec((_TB, hist, feat), lambda i: (i, 0, 0)),
        out_shape=jax.ShapeDtypeStruct((batch, hist, feat), jnp.bfloat16),
    )


def kernel(inputs, embedding):
    batch, hist = inputs.shape
    n_emb, feat = embedding.shape
    idx = inputs.astype(jnp.int32)
    g = _build_gather(batch, hist, n_emb, feat)(embedding, idx)
    return _build_extract(batch, hist, feat)(idx, g)


# speedup vs baseline: 1.4796x; 1.0189x over previous
import functools

import jax
import jax.numpy as jnp
from jax import lax
from jax.experimental import pallas as pl
from jax.experimental.pallas import tpu as pltpu
from jax.experimental.pallas import tpu_sc as plsc

_NC = 2
_NS = 16
_NW = _NC * _NS
_K = 2
_NBUF = 4
_TB = 256


@functools.cache
def _build_gather(batch: int, hist: int, n_emb: int, feat: int):
    rpw = batch // _NW
    nblk = rpw // _K
    assert batch == _NW * rpw and rpw % _K == 0 and n_emb % 2 == 0
    assert nblk % _NBUF == 0 and nblk // _NBUF >= 2 and 16 <= hist <= 128
    mesh = plsc.VectorSubcoreMesh(core_axis_name="c", subcore_axis_name="s")

    @functools.partial(
        pl.kernel,
        out_type=jax.ShapeDtypeStruct((batch, hist, feat), jnp.int32),
        mesh=mesh,
        scratch_types=[
            pltpu.VMEM((rpw, hist), jnp.int32),
            pltpu.VMEM((rpw, hist), jnp.int32),
        ]
        + [pltpu.VMEM((_K, hist, feat), jnp.int32) for _ in range(_NBUF)]
        + [pltpu.SemaphoreType.DMA] * (2 * _NBUF),
        compiler_params=pltpu.CompilerParams(use_tc_tiling_on_sc=True),
    )
    def gather_kernel(table_hbm, idx_hbm, g_hbm, idx_v, rind_v, *rest):
        tab32 = table_hbm.bitcast(jnp.int32)
        gbufs = rest[:_NBUF]
        gsems = rest[_NBUF:2 * _NBUF]
        ssems = rest[2 * _NBUF:]
        wid = lax.axis_index("s") * _NC + lax.axis_index("c")
        row0 = wid * rpw
        pltpu.sync_copy(idx_hbm.at[pl.ds(row0, rpw)], idx_v)

        offs = list(range(0, hist - 15, 16))
        if offs[-1] != hist - 16:
            offs.append(hist - 16)

        def prep_body(b, carry):
            for off in offs:
                rind_v[b, pl.ds(off, 16)] = idx_v[b, pl.ds(off, 16)] >> 1
            return carry

        lax.fori_loop(0, rpw, prep_body, 0)

        def fire(t, b):
            for k in range(_K):
                pltpu.async_copy(
                    tab32.at[rind_v.at[t * _K + k]], gbufs[b].at[k], gsems[b]
                )

        def drain_g(b):
            pltpu.make_async_copy(
                g_hbm.at[pl.ds(0, _K)], gbufs[b], gsems[b]
            ).wait()

        def store_o(t, b):
            pltpu.async_copy(
                gbufs[b], g_hbm.at[pl.ds(row0 + t * _K, _K)], ssems[b]
            )

        def wait_s(b):
            pltpu.make_async_copy(
                gbufs[b], g_hbm.at[pl.ds(0, _K)], ssems[b]
            ).wait()

        for b in range(_NBUF):
            fire(b, b)

        def body(i, carry):
            for b in range(_NBUF):
                drain_g(b)
                store_o(i * _NBUF + b, b)
            for b in range(_NBUF):
                wait_s(b)
                fire(i * _NBUF + b + _NBUF, b)
            return carry

        lax.fori_loop(0, nblk // _NBUF - 1, body, 0)

        t0 = nblk - _NBUF
        for b in range(_NBUF):
            drain_g(b)
            store_o(t0 + b, b)
        for b in range(_NBUF):
            wait_s(b)

    return gather_kernel


def _extract_body(idx_ref, g_ref, o_ref):
    sh = (idx_ref[...] & 1) << 4
    v = (g_ref[...] >> sh[:, :, None]) & 0xFFFF
    o_ref[...] = lax.bitcast_convert_type(
        v.astype(jnp.uint16), jnp.bfloat16
    )


@functools.cache
def _build_extract(batch: int, hist: int, feat: int):
    assert batch % _TB == 0
    return pl.pallas_call(
        _extract_body,
        grid=(batch // _TB,),
        in_specs=[
            pl.BlockSpec((_TB, hist), lambda i: (i, 0)),
            pl.BlockSpec((_TB, hist, feat), lambda i: (i, 0, 0)),
        ],
        out_specs=pl.BlockSpec((_TB, hist, feat), lambda i: (i, 0, 0)),
        out_shape=jax.ShapeDtypeStruct((batch, hist, feat), jnp.bfloat16),
    )


def kernel(inputs, embedding):
    batch, hist = inputs.shape
    n_emb, feat = embedding.shape
    idx = inputs.astype(jnp.int32)
    g = _build_gather(batch, hist, n_emb, feat)(embedding, idx)
    return _build_extract(batch, hist, feat)(idx, g)
